# Initial kernel scaffold; baseline (speedup 1.0000x reference)
import jax
import jax.numpy as jnp
from jax.experimental import pallas as pl


def kernel(x, edge_index, W0, b0, W1, b1, W2, b2):
    raise NotImplementedError("write your pallas kernel here")


# speedup vs baseline: 9.8033x; 9.8033x over previous
import functools

import jax
import jax.numpy as jnp
from jax import lax
from jax.experimental import pallas as pl
from jax.experimental.pallas import tpu as pltpu
from jax.experimental.pallas import tpu_sc as plsc

N_NODES = 50000
NPAD = 50176
HALF = 25088
ACC_ROWS = 25216
TRASH = HALF
E_EDGES = 800000
CHUNK = 128
N_CHUNKS = 6256
E_PAD = N_CHUNKS * CHUNK
Q_PER_TILE = N_CHUNKS // 16
ZSTRIPE = ACC_ROWS // 16
OSTRIPE = HALF // 16

_MESH = plsc.VectorSubcoreMesh(core_axis_name="c", subcore_axis_name="s")
_SC_PARAMS = pltpu.CompilerParams(use_tc_tiling_on_sc=False)


def _remap_dst(didx_ref, lidx_ref, c_lo):
    for j in range(CHUNK // 16):
        d = didx_ref[pl.ds(j * 16, 16)]
        loc = d - c_lo
        oob = (loc < 0) | (loc >= HALF)
        spread = TRASH + (d & 127)
        lidx_ref[pl.ds(j * 16, 16)] = jnp.where(oob, spread, loc)


def _make_deg_kernel():
    @functools.partial(
        pl.kernel,
        out_type=jax.ShapeDtypeStruct((NPAD, 16), jnp.float32),
        mesh=_MESH,
        compiler_params=_SC_PARAMS,
        scratch_types=[
            pltpu.VMEM((CHUNK,), jnp.int32),
            pltpu.VMEM((CHUNK,), jnp.int32),
            pltpu.VMEM((CHUNK, 16), jnp.float32),
            pltpu.VMEM_SHARED((ACC_ROWS, 16), jnp.float32),
        ],
    )
    def deg_kernel(dst_hbm, ones_hbm, zeros_hbm, out_hbm, didx, lidx, ones_v, acc):
        cid = lax.axis_index("c")
        sid = lax.axis_index("s")
        c_lo = cid * HALF
        zbase = sid * ZSTRIPE
        pltpu.sync_copy(zeros_hbm.at[pl.ds(0, ZSTRIPE)], acc.at[pl.ds(zbase, ZSTRIPE)])
        pltpu.sync_copy(ones_hbm, ones_v)
        plsc.subcore_barrier()

        def body(i, _):
            c = sid * Q_PER_TILE + i
            pltpu.sync_copy(dst_hbm.at[pl.ds(c * CHUNK, CHUNK)], didx)
            _remap_dst(didx, lidx, c_lo)
            pltpu.sync_copy(ones_v, acc.at[lidx], add=True)
            return _

        lax.fori_loop(0, Q_PER_TILE, body, None)
        plsc.subcore_barrier()
        obase = sid * OSTRIPE
        pltpu.sync_copy(acc.at[pl.ds(obase, OSTRIPE)],
                        out_hbm.at[pl.ds(c_lo + obase, OSTRIPE)])

    return deg_kernel


def _make_agg_kernel(depth):
    @functools.partial(
        pl.kernel,
        out_type=jax.ShapeDtypeStruct((NPAD, depth), jnp.float32),
        mesh=_MESH,
        compiler_params=_SC_PARAMS,
        scratch_types=[
            pltpu.VMEM((CHUNK,), jnp.int32),
            pltpu.VMEM((CHUNK,), jnp.int32),
            pltpu.VMEM((CHUNK,), jnp.int32),
            pltpu.VMEM((CHUNK, depth), jnp.float32),
            pltpu.VMEM_SHARED((ACC_ROWS, depth), jnp.float32),
        ],
    )
    def agg_kernel(src_hbm, dst_hbm, g_hbm, zeros_hbm, out_hbm,
                   sidx, didx, lidx, rows, acc):
        cid = lax.axis_index("c")
        sid = lax.axis_index("s")
        c_lo = cid * HALF
        zbase = sid * ZSTRIPE
        pltpu.sync_copy(zeros_hbm.at[pl.ds(0, ZSTRIPE)], acc.at[pl.ds(zbase, ZSTRIPE)])
        plsc.subcore_barrier()

        def body(i, _):
            c = sid * Q_PER_TILE + i
            pltpu.sync_copy(src_hbm.at[pl.ds(c * CHUNK, CHUNK)], sidx)
            pltpu.sync_copy(dst_hbm.at[pl.ds(c * CHUNK, CHUNK)], didx)
            _remap_dst(didx, lidx, c_lo)
            pltpu.sync_copy(g_hbm.at[sidx], rows)
            pltpu.sync_copy(rows, acc.at[lidx], add=True)
            return _

        lax.fori_loop(0, Q_PER_TILE, body, None)
        plsc.subcore_barrier()
        obase = sid * OSTRIPE
        pltpu.sync_copy(acc.at[pl.ds(obase, OSTRIPE)],
                        out_hbm.at[pl.ds(c_lo + obase, OSTRIPE)])

    return agg_kernel


_RB = NPAD // 8


def _dinv(deg_blk):
    return lax.rsqrt(deg_blk[:, 0:1] + 1.0)


def _tc_call(body, n_in_feats, d_out, extra_specs):
    grid = (8,)
    return pl.pallas_call(
        body,
        grid=grid,
        in_specs=extra_specs,
        out_specs=pl.BlockSpec((_RB, d_out), lambda i: (i, 0)),
        out_shape=jax.ShapeDtypeStruct((NPAD, d_out), jnp.float32),
    )


def _stage0(x, W0, deg):
    def body(x_ref, w_ref, deg_ref, o_ref):
        o_ref[:, :] = jnp.dot(x_ref[:, :], w_ref[:, :],
                              preferred_element_type=jnp.float32) * _dinv(deg_ref)

    specs = [
        pl.BlockSpec((_RB, 16), lambda i: (i, 0)),
        pl.BlockSpec((16, 64), lambda i: (0, 0)),
        pl.BlockSpec((_RB, 16), lambda i: (i, 0)),
    ]
    return _tc_call(body, 16, 64, specs)(x, W0, deg)


def _stage_mid(agg, gh, deg, b, W, d_in, d_out):
    def body(agg_ref, gh_ref, deg_ref, b_ref, w_ref, o_ref):
        dinv = _dinv(deg_ref)
        h = jnp.maximum(dinv * (agg_ref[:, :] + gh_ref[:, :]) + b_ref[0:1, :], 0.0)
        o_ref[:, :] = jnp.dot(h, w_ref[:, :],
                              preferred_element_type=jnp.float32) * dinv

    specs = [
        pl.BlockSpec((_RB, d_in), lambda i: (i, 0)),
        pl.BlockSpec((_RB, d_in), lambda i: (i, 0)),
        pl.BlockSpec((_RB, 16), lambda i: (i, 0)),
        pl.BlockSpec((1, d_in), lambda i: (0, 0)),
        pl.BlockSpec((d_in, d_out), lambda i: (0, 0)),
    ]
    return _tc_call(body, d_in, d_out, specs)(agg, gh, deg, b, W)


def _stage_final(agg, gh, deg, b):
    def body(agg_ref, gh_ref, deg_ref, b_ref, o_ref):
        o_ref[:, :] = _dinv(deg_ref) * (agg_ref[:, :] + gh_ref[:, :]) + b_ref[0:1, :]

    specs = [
        pl.BlockSpec((_RB, 16), lambda i: (i, 0)),
        pl.BlockSpec((_RB, 16), lambda i: (i, 0)),
        pl.BlockSpec((_RB, 16), lambda i: (i, 0)),
        pl.BlockSpec((1, 16), lambda i: (0, 0)),
    ]
    return _tc_call(body, 16, 16, specs)(agg, gh, deg, b)


def kernel(x, edge_index, W0, b0, W1, b1, W2, b2):
    src = edge_index[0].astype(jnp.int32)
    dst = edge_index[1].astype(jnp.int32)
    pad = E_PAD - E_EDGES
    src_p = jnp.concatenate([src, jnp.zeros((pad,), jnp.int32)])
    dst_p = jnp.concatenate([dst, jnp.full((pad,), 1 << 28, jnp.int32)])

    x_p = jnp.pad(x, ((0, NPAD - N_NODES), (0, 0)))
    W2p = jnp.pad(W2, ((0, 0), (0, 1)))
    b2p = jnp.pad(b2, (0, 1)).reshape(1, 16)
    b0r = b0.reshape(1, 64)
    b1r = b1.reshape(1, 64)

    e0_rows = jnp.zeros((CHUNK, 16), jnp.float32).at[:, 0].set(1.0)
    zeros16 = jnp.zeros((ZSTRIPE, 16), jnp.float32)
    zeros64 = jnp.zeros((ZSTRIPE, 64), jnp.float32)

    deg = _make_deg_kernel()(dst_p, e0_rows, zeros16)

    agg64 = _make_agg_kernel(64)
    agg16 = _make_agg_kernel(16)

    gh0 = _stage0(x_p, W0, deg)
    a0 = agg64(src_p, dst_p, gh0, zeros64)
    gh1 = _stage_mid(a0, gh0, deg, b0r, W1, 64, 64)
    a1 = agg64(src_p, dst_p, gh1, zeros64)
    gh2 = _stage_mid(a1, gh1, deg, b1r, W2p, 64, 16)
    a2 = agg16(src_p, dst_p, gh2, zeros16)
    out = _stage_final(a2, gh2, deg, b2p)
    return out[:N_NODES, :15]


# speedup vs baseline: 20.7641x; 2.1181x over previous
import functools

import jax
import jax.numpy as jnp
from jax import lax
from jax.experimental import pallas as pl
from jax.experimental.pallas import tpu as pltpu
from jax.experimental.pallas import tpu_sc as plsc

N_NODES = 50000
NPAD = 50176
HALF = 25088
ACC_ROWS = 25216
TRASH = HALF
E_EDGES = 800000
CHUNK = 128
N_CHUNKS = 6272
E_PAD = N_CHUNKS * CHUNK
Q_PER_TILE = N_CHUNKS // 16
ZSTRIPE = ACC_ROWS // 16
OSTRIPE = HALF // 16

_MESH = plsc.VectorSubcoreMesh(core_axis_name="c", subcore_axis_name="s")
_SC_PARAMS = pltpu.CompilerParams(use_tc_tiling_on_sc=False)


def _remap(d, c_lo):
    loc = d - c_lo
    oob = (loc < 0) | (loc >= HALF)
    return jnp.where(oob, TRASH + (d & 127), loc)


def _make_deg_kernel():
    @functools.partial(
        pl.kernel,
        out_type=jax.ShapeDtypeStruct((NPAD, 16), jnp.float32),
        mesh=_MESH,
        compiler_params=_SC_PARAMS,
        scratch_types=[
            pltpu.VMEM((2, CHUNK), jnp.int32),
            pltpu.VMEM((2, CHUNK), jnp.int32),
            pltpu.VMEM((CHUNK, 16), jnp.float32),
            [pltpu.SemaphoreType.DMA] * 2,
            pltpu.VMEM_SHARED((ACC_ROWS, 16), jnp.float32),
        ],
    )
    def deg_kernel(dst_hbm, ones_hbm, zeros_hbm, out_hbm,
                   didx, lidx, ones_v, isems, acc):
        cid = lax.axis_index("c")
        sid = lax.axis_index("s")
        c_lo = cid * HALF
        zbase = sid * ZSTRIPE
        pltpu.sync_copy(zeros_hbm.at[pl.ds(0, ZSTRIPE)], acc.at[pl.ds(zbase, ZSTRIPE)])
        pltpu.sync_copy(ones_hbm, ones_v)
        plsc.subcore_barrier()
        base = sid * Q_PER_TILE

        def issue_idx(c, b):
            pltpu.async_copy(dst_hbm.at[pl.ds((base + c) * CHUNK, CHUNK)],
                             didx.at[b], isems[b])

        def wait_idx(c, b):
            pltpu.make_async_copy(dst_hbm.at[pl.ds((base + c) * CHUNK, CHUNK)],
                                  didx.at[b], isems[b]).wait()

        def process(c, b):
            wait_idx(c, b)
            for j in range(CHUNK // 16):
                lidx[b, pl.ds(j * 16, 16)] = _remap(didx[b, pl.ds(j * 16, 16)], c_lo)
            pltpu.sync_copy(ones_v, acc.at[lidx.at[b]], add=True)

        issue_idx(0, 0)

        def body(k, _):
            for j in range(2):
                c = 2 * k + j
                issue_idx(c + 1, (j + 1) % 2)
                process(c, j)
            return _

        lax.fori_loop(0, (Q_PER_TILE - 2) // 2, body, None)
        issue_idx(Q_PER_TILE - 1, 1)
        process(Q_PER_TILE - 2, 0)
        process(Q_PER_TILE - 1, 1)

        plsc.subcore_barrier()
        obase = sid * OSTRIPE
        pltpu.sync_copy(acc.at[pl.ds(obase, OSTRIPE)],
                        out_hbm.at[pl.ds(c_lo + obase, OSTRIPE)])

    return deg_kernel


def _make_agg_kernel(depth):
    @functools.partial(
        pl.kernel,
        out_type=jax.ShapeDtypeStruct((NPAD, depth), jnp.float32),
        mesh=_MESH,
        compiler_params=_SC_PARAMS,
        scratch_types=[
            pltpu.VMEM((3, 2 * CHUNK), jnp.int32),
            pltpu.VMEM((3, CHUNK), jnp.int32),
            pltpu.VMEM((3, CHUNK, depth), jnp.float32),
            [pltpu.SemaphoreType.DMA] * 3,
            [pltpu.SemaphoreType.DMA] * 3,
            pltpu.VMEM_SHARED((ACC_ROWS, depth), jnp.float32),
        ],
    )
    def agg_kernel(epk_hbm, g_hbm, zeros_hbm, out_hbm,
                   sd, lidx, rows, isems, gsems, acc):
        cid = lax.axis_index("c")
        sid = lax.axis_index("s")
        c_lo = cid * HALF
        zbase = sid * ZSTRIPE
        pltpu.sync_copy(zeros_hbm.at[pl.ds(0, ZSTRIPE)], acc.at[pl.ds(zbase, ZSTRIPE)])
        plsc.subcore_barrier()
        base = sid * Q_PER_TILE

        def issue_idx(c, b):
            pltpu.async_copy(epk_hbm.at[pl.ds((base + c) * 2 * CHUNK, 2 * CHUNK)],
                             sd.at[b], isems[b])

        def wait_idx(c, b):
            pltpu.make_async_copy(epk_hbm.at[pl.ds((base + c) * 2 * CHUNK, 2 * CHUNK)],
                                  sd.at[b], isems[b]).wait()

        def start_chunk(c, b):
            wait_idx(c, b)
            for j in range(CHUNK // 16):
                lidx[b, pl.ds(j * 16, 16)] = _remap(
                    sd[b, pl.ds(CHUNK + j * 16, 16)], c_lo)
            pltpu.async_copy(g_hbm.at[sd.at[b, pl.ds(0, CHUNK)]], rows.at[b],
                             gsems[b])

        def finish_chunk(b):
            pltpu.make_async_copy(g_hbm.at[sd.at[b, pl.ds(0, CHUNK)]], rows.at[b],
                                  gsems[b]).wait()
            pltpu.sync_copy(rows.at[b], acc.at[lidx.at[b]], add=True)

        issue_idx(0, 0)
        issue_idx(1, 1)
        start_chunk(0, 0)

        def body(k, _):
            for j in range(3):
                c = 3 * k + j
                issue_idx(c + 2, (j + 2) % 3)
                start_chunk(c + 1, (j + 1) % 3)
                finish_chunk(j)
            return _

        lax.fori_loop(0, (Q_PER_TILE - 2) // 3, body, None)
        bq2 = (Q_PER_TILE - 2) % 3
        bq1 = (Q_PER_TILE - 1) % 3
        start_chunk(Q_PER_TILE - 1, bq1)
        finish_chunk(bq2)
        finish_chunk(bq1)

        plsc.subcore_barrier()
        obase = sid * OSTRIPE
        pltpu.sync_copy(acc.at[pl.ds(obase, OSTRIPE)],
                        out_hbm.at[pl.ds(c_lo + obase, OSTRIPE)])

    return agg_kernel


_RB = NPAD // 8


def _dinv(deg_blk):
    return lax.rsqrt(deg_blk[:, 0:1] + 1.0)


def _tc_call(body, d_out, extra_specs):
    return pl.pallas_call(
        body,
        grid=(8,),
        in_specs=extra_specs,
        out_specs=pl.BlockSpec((_RB, d_out), lambda i: (i, 0)),
        out_shape=jax.ShapeDtypeStruct((NPAD, d_out), jnp.float32),
    )


def _stage0(x, W0, deg):
    def body(x_ref, w_ref, deg_ref, o_ref):
        o_ref[:, :] = jnp.dot(x_ref[:, :], w_ref[:, :],
                              preferred_element_type=jnp.float32) * _dinv(deg_ref)

    specs = [
        pl.BlockSpec((_RB, 16), lambda i: (i, 0)),
        pl.BlockSpec((16, 64), lambda i: (0, 0)),
        pl.BlockSpec((_RB, 16), lambda i: (i, 0)),
    ]
    return _tc_call(body, 64, specs)(x, W0, deg)


def _stage_mid(agg, gh, deg, b, W, d_in, d_out):
    def body(agg_ref, gh_ref, deg_ref, b_ref, w_ref, o_ref):
        dinv = _dinv(deg_ref)
        h = jnp.maximum(dinv * (agg_ref[:, :] + gh_ref[:, :]) + b_ref[0:1, :], 0.0)
        o_ref[:, :] = jnp.dot(h, w_ref[:, :],
                              preferred_element_type=jnp.float32) * dinv

    specs = [
        pl.BlockSpec((_RB, d_in), lambda i: (i, 0)),
        pl.BlockSpec((_RB, d_in), lambda i: (i, 0)),
        pl.BlockSpec((_RB, 16), lambda i: (i, 0)),
        pl.BlockSpec((1, d_in), lambda i: (0, 0)),
        pl.BlockSpec((d_in, d_out), lambda i: (0, 0)),
    ]
    return _tc_call(body, d_out, specs)(agg, gh, deg, b, W)


def _stage_final(agg, gh, deg, b):
    def body(agg_ref, gh_ref, deg_ref, b_ref, o_ref):
        o_ref[:, :] = _dinv(deg_ref) * (agg_ref[:, :] + gh_ref[:, :]) + b_ref[0:1, :]

    specs = [
        pl.BlockSpec((_RB, 16), lambda i: (i, 0)),
        pl.BlockSpec((_RB, 16), lambda i: (i, 0)),
        pl.BlockSpec((_RB, 16), lambda i: (i, 0)),
        pl.BlockSpec((1, 16), lambda i: (0, 0)),
    ]
    return _tc_call(body, 16, specs)(agg, gh, deg, b)


def kernel(x, edge_index, W0, b0, W1, b1, W2, b2):
    src = edge_index[0].astype(jnp.int32)
    dst = edge_index[1].astype(jnp.int32)
    pad = E_PAD - E_EDGES
    src_p = jnp.concatenate([src, jnp.zeros((pad,), jnp.int32)])
    dst_p = jnp.concatenate([dst, jnp.full((pad,), 1 << 28, jnp.int32)])
    epk = jnp.stack([src_p.reshape(N_CHUNKS, CHUNK),
                     dst_p.reshape(N_CHUNKS, CHUNK)], axis=1).reshape(-1)

    x_p = jnp.pad(x, ((0, NPAD - N_NODES), (0, 0)))
    W2p = jnp.pad(W2, ((0, 0), (0, 1)))
    b2p = jnp.pad(b2, (0, 1)).reshape(1, 16)
    b0r = b0.reshape(1, 64)
    b1r = b1.reshape(1, 64)

    e0_rows = jnp.zeros((CHUNK, 16), jnp.float32).at[:, 0].set(1.0)
    zeros16 = jnp.zeros((ZSTRIPE, 16), jnp.float32)
    zeros64 = jnp.zeros((ZSTRIPE, 64), jnp.float32)

    deg = _make_deg_kernel()(dst_p, e0_rows, zeros16)

    agg64 = _make_agg_kernel(64)
    agg16 = _make_agg_kernel(16)

    gh0 = _stage0(x_p, W0, deg)
    a0 = agg64(epk, gh0, zeros64)
    gh1 = _stage_mid(a0, gh0, deg, b0r, W1, 64, 64)
    a1 = agg64(epk, gh1, zeros64)
    gh2 = _stage_mid(a1, gh1, deg, b1r, W2p, 64, 16)
    a2 = agg16(epk, gh2, zeros16)
    out = _stage_final(a2, gh2, deg, b2p)
    return out[:N_NODES, :15]


# speedup vs baseline: 22.5241x; 1.0848x over previous
import functools

import jax
import jax.numpy as jnp
from jax import lax
from jax.experimental import pallas as pl
from jax.experimental.pallas import tpu as pltpu
from jax.experimental.pallas import tpu_sc as plsc

N_NODES = 50000
NPAD = 50176
HALF = 25088
ACC_ROWS = 25216
TRASH = HALF
E_EDGES = 800000
CHUNK = 128
N_CHUNKS = 6272
E_PAD = N_CHUNKS * CHUNK
Q_PER_TILE = N_CHUNKS // 16
ZSTRIPE = ACC_ROWS // 16
OSTRIPE = HALF // 16

_MESH = plsc.VectorSubcoreMesh(core_axis_name="c", subcore_axis_name="s")
_SC_PARAMS = pltpu.CompilerParams(use_tc_tiling_on_sc=False)


def _remap(d, c_lo):
    loc = d - c_lo
    oob = (loc < 0) | (loc >= HALF)
    return jnp.where(oob, TRASH + (d & 127), loc)


def _make_deg_kernel():
    @functools.partial(
        pl.kernel,
        out_type=jax.ShapeDtypeStruct((NPAD, 8), jnp.float32),
        mesh=_MESH,
        compiler_params=_SC_PARAMS,
        scratch_types=[
            pltpu.VMEM((2, CHUNK), jnp.int32),
            pltpu.VMEM((2, CHUNK), jnp.int32),
            pltpu.VMEM((CHUNK, 8), jnp.float32),
            [pltpu.SemaphoreType.DMA] * 2,
            pltpu.VMEM_SHARED((ACC_ROWS, 8), jnp.float32),
        ],
    )
    def deg_kernel(dst_hbm, ones_hbm, zeros_hbm, out_hbm,
                   didx, lidx, ones_v, isems, acc):
        cid = lax.axis_index("c")
        sid = lax.axis_index("s")
        c_lo = cid * HALF
        zbase = sid * ZSTRIPE
        pltpu.sync_copy(zeros_hbm.at[pl.ds(0, ZSTRIPE)], acc.at[pl.ds(zbase, ZSTRIPE)])
        pltpu.sync_copy(ones_hbm, ones_v)
        plsc.subcore_barrier()
        base = sid * Q_PER_TILE

        def issue_idx(c, b):
            pltpu.async_copy(dst_hbm.at[pl.ds((base + c) * CHUNK, CHUNK)],
                             didx.at[b], isems[b])

        def wait_idx(c, b):
            pltpu.make_async_copy(dst_hbm.at[pl.ds((base + c) * CHUNK, CHUNK)],
                                  didx.at[b], isems[b]).wait()

        def process(c, b):
            wait_idx(c, b)
            for j in range(CHUNK // 16):
                lidx[b, pl.ds(j * 16, 16)] = _remap(didx[b, pl.ds(j * 16, 16)], c_lo)
            pltpu.sync_copy(ones_v, acc.at[lidx.at[b]], add=True)

        issue_idx(0, 0)

        def body(k, _):
            for j in range(2):
                c = 2 * k + j
                issue_idx(c + 1, (j + 1) % 2)
                process(c, j)
            return _

        lax.fori_loop(0, (Q_PER_TILE - 2) // 2, body, None)
        issue_idx(Q_PER_TILE - 1, 1)
        process(Q_PER_TILE - 2, 0)
        process(Q_PER_TILE - 1, 1)

        plsc.subcore_barrier()
        obase = sid * OSTRIPE
        pltpu.sync_copy(acc.at[pl.ds(obase, OSTRIPE)],
                        out_hbm.at[pl.ds(c_lo + obase, OSTRIPE)])

    return deg_kernel


def _make_agg_kernel(depth):
    @functools.partial(
        pl.kernel,
        out_type=jax.ShapeDtypeStruct((NPAD, depth), jnp.float32),
        mesh=_MESH,
        compiler_params=_SC_PARAMS,
        scratch_types=[
            pltpu.VMEM((3, CHUNK), jnp.int32),
            pltpu.VMEM((3, CHUNK), jnp.int32),
            pltpu.VMEM((3, CHUNK), jnp.int32),
            pltpu.VMEM((3, CHUNK, depth), jnp.float32),
            [pltpu.SemaphoreType.DMA] * 3,
            [pltpu.SemaphoreType.DMA] * 3,
            [pltpu.SemaphoreType.DMA] * 3,
            [pltpu.SemaphoreType.DMA] * 3,
            pltpu.VMEM_SHARED((ACC_ROWS, depth), jnp.float32),
        ],
    )
    def agg_kernel(src_hbm, dst_hbm, g_hbm, zeros_hbm, out_hbm,
                   sidx, didx, lidx, rows, isems, jsems, gsems, ssems, acc):
        cid = lax.axis_index("c")
        sid = lax.axis_index("s")
        c_lo = cid * HALF
        zbase = sid * ZSTRIPE
        pltpu.sync_copy(zeros_hbm.at[pl.ds(0, ZSTRIPE)], acc.at[pl.ds(zbase, ZSTRIPE)])
        plsc.subcore_barrier()
        base = sid * Q_PER_TILE

        def issue_idx(c, b):
            pltpu.async_copy(src_hbm.at[pl.ds((base + c) * CHUNK, CHUNK)],
                             sidx.at[b], isems[b])
            pltpu.async_copy(dst_hbm.at[pl.ds((base + c) * CHUNK, CHUNK)],
                             didx.at[b], jsems[b])

        def wait_idx(c, b):
            pltpu.make_async_copy(src_hbm.at[pl.ds((base + c) * CHUNK, CHUNK)],
                                  sidx.at[b], isems[b]).wait()
            pltpu.make_async_copy(dst_hbm.at[pl.ds((base + c) * CHUNK, CHUNK)],
                                  didx.at[b], jsems[b]).wait()

        def wait_scat(b):
            pltpu.make_async_copy(rows.at[b], acc.at[lidx.at[b]], ssems[b]).wait()

        def start_chunk(c, b, guard):
            wait_idx(c, b)
            if guard is True:
                wait_scat(b)
            elif guard is not False:
                @pl.when(guard)
                def _():
                    wait_scat(b)
            for j in range(CHUNK // 16):
                lidx[b, pl.ds(j * 16, 16)] = _remap(
                    didx[b, pl.ds(j * 16, 16)], c_lo)
            pltpu.async_copy(g_hbm.at[sidx.at[b]], rows.at[b], gsems[b])

        def finish_chunk(b):
            pltpu.make_async_copy(g_hbm.at[sidx.at[b]], rows.at[b],
                                  gsems[b]).wait()
            pltpu.async_copy(rows.at[b], acc.at[lidx.at[b]], ssems[b], add=True)

        issue_idx(0, 0)
        issue_idx(1, 1)
        start_chunk(0, 0, False)

        def body(k, _):
            for j in range(3):
                c = 3 * k + j
                issue_idx(c + 2, (j + 2) % 3)
                start_chunk(c + 1, (j + 1) % 3, True if j == 2 else (k > 0))
                finish_chunk(j)
            return _

        lax.fori_loop(0, (Q_PER_TILE - 2) // 3, body, None)
        bq2 = (Q_PER_TILE - 2) % 3
        bq1 = (Q_PER_TILE - 1) % 3
        start_chunk(Q_PER_TILE - 1, bq1, True)
        finish_chunk(bq2)
        finish_chunk(bq1)
        wait_scat((Q_PER_TILE - 3) % 3)
        wait_scat(bq2)
        wait_scat(bq1)

        plsc.subcore_barrier()
        obase = sid * OSTRIPE
        pltpu.sync_copy(acc.at[pl.ds(obase, OSTRIPE)],
                        out_hbm.at[pl.ds(c_lo + obase, OSTRIPE)])

    return agg_kernel


_RB = NPAD // 8


def _dinv(deg_blk):
    return lax.rsqrt(deg_blk[:, 0:1] + 1.0)


def _tc_call(body, d_out, extra_specs):
    return pl.pallas_call(
        body,
        grid=(8,),
        in_specs=extra_specs,
        out_specs=pl.BlockSpec((_RB, d_out), lambda i: (i, 0)),
        out_shape=jax.ShapeDtypeStruct((NPAD, d_out), jnp.float32),
    )


def _stage0(x, W0, deg):
    def body(x_ref, w_ref, deg_ref, o_ref):
        o_ref[:, :] = jnp.dot(x_ref[:, :], w_ref[:, :],
                              preferred_element_type=jnp.float32) * _dinv(deg_ref)

    specs = [
        pl.BlockSpec((_RB, 16), lambda i: (i, 0)),
        pl.BlockSpec((16, 64), lambda i: (0, 0)),
        pl.BlockSpec((_RB, 8), lambda i: (i, 0)),
    ]
    return _tc_call(body, 64, specs)(x, W0, deg)


def _stage_mid(agg, gh, deg, b, W, d_in, d_out):
    def body(agg_ref, gh_ref, deg_ref, b_ref, w_ref, o_ref):
        dinv = _dinv(deg_ref)
        h = jnp.maximum(dinv * (agg_ref[:, :] + gh_ref[:, :]) + b_ref[0:1, :], 0.0)
        o_ref[:, :] = jnp.dot(h, w_ref[:, :],
                              preferred_element_type=jnp.float32) * dinv

    specs = [
        pl.BlockSpec((_RB, d_in), lambda i: (i, 0)),
        pl.BlockSpec((_RB, d_in), lambda i: (i, 0)),
        pl.BlockSpec((_RB, 8), lambda i: (i, 0)),
        pl.BlockSpec((1, d_in), lambda i: (0, 0)),
        pl.BlockSpec((d_in, d_out), lambda i: (0, 0)),
    ]
    return _tc_call(body, d_out, specs)(agg, gh, deg, b, W)


def _stage_final(agg, gh, deg, b):
    def body(agg_ref, gh_ref, deg_ref, b_ref, o_ref):
        o_ref[:, :] = _dinv(deg_ref) * (agg_ref[:, :] + gh_ref[:, :]) + b_ref[0:1, :]

    specs = [
        pl.BlockSpec((_RB, 16), lambda i: (i, 0)),
        pl.BlockSpec((_RB, 16), lambda i: (i, 0)),
        pl.BlockSpec((_RB, 8), lambda i: (i, 0)),
        pl.BlockSpec((1, 16), lambda i: (0, 0)),
    ]
    return _tc_call(body, 16, specs)(agg, gh, deg, b)


def kernel(x, edge_index, W0, b0, W1, b1, W2, b2):
    src = edge_index[0].astype(jnp.int32)
    dst = edge_index[1].astype(jnp.int32)
    pad = E_PAD - E_EDGES
    src_p = jnp.concatenate([src, jnp.zeros((pad,), jnp.int32)])
    dst_p = jnp.concatenate([dst, jnp.full((pad,), 1 << 28, jnp.int32)])

    x_p = jnp.pad(x, ((0, NPAD - N_NODES), (0, 0)))
    W2p = jnp.pad(W2, ((0, 0), (0, 1)))
    b2p = jnp.pad(b2, (0, 1)).reshape(1, 16)
    b0r = b0.reshape(1, 64)
    b1r = b1.reshape(1, 64)

    e0_rows = jnp.zeros((CHUNK, 8), jnp.float32).at[:, 0].set(1.0)
    zeros8 = jnp.zeros((ZSTRIPE, 8), jnp.float32)
    zeros16 = jnp.zeros((ZSTRIPE, 16), jnp.float32)
    zeros64 = jnp.zeros((ZSTRIPE, 64), jnp.float32)

    deg = _make_deg_kernel()(dst_p, e0_rows, zeros8)

    agg64 = _make_agg_kernel(64)
    agg16 = _make_agg_kernel(16)

    gh0 = _stage0(x_p, W0, deg)
    a0 = agg64(src_p, dst_p, gh0, zeros64)
    gh1 = _stage_mid(a0, gh0, deg, b0r, W1, 64, 64)
    a1 = agg64(src_p, dst_p, gh1, zeros64)
    gh2 = _stage_mid(a1, gh1, deg, b1r, W2p, 64, 16)
    a2 = agg16(src_p, dst_p, gh2, zeros16)
    out = _stage_final(a2, gh2, deg, b2p)
    return out[:N_NODES, :15]
